# Initial kernel scaffold; baseline (speedup 1.0000x reference)
import jax
import jax.numpy as jnp
from jax.experimental import pallas as pl


def kernel(x, edge_attr, edge_weight, params, edge_index, batch, inter_graph_idx):
    raise NotImplementedError("write your pallas kernel here")


# speedup vs baseline: 2.0758x; 2.0758x over previous
import functools

import jax
import jax.numpy as jnp
from jax import lax
from jax.experimental import pallas as pl
from jax.experimental.pallas import tpu as pltpu
from jax.experimental.pallas import tpu_sc as plsc

N = 10000
E = 320000
DIM = 128
NG = 64
NS_GRAPH = 8

SC_CORES = 2
SC_SUBCORES = 16
NTILES = SC_CORES * SC_SUBCORES
EPT = E // NTILES
CHUNK = 80
NCHUNK = EPT // CHUNK
N_PAD = 10240
ROWS_PT = N_PAD // SC_SUBCORES
ZROWS = 128


def _make_mp(d):
    mesh = plsc.VectorSubcoreMesh(core_axis_name="c", subcore_axis_name="s")

    def body(h_hbm, emb_hbm, src_hbm, dst_hbm, ew_hbm, out_hbm,
             acc, dst_v, src_c, ew_c, hrows, embv, sem):
        cid = lax.axis_index("c")
        sid = lax.axis_index("s")
        tid = cid * SC_SUBCORES + sid

        def zrow(i, carry):
            for k in range(d // 16):
                hrows[i, pl.ds(k * 16, 16)] = jnp.zeros((16,), jnp.float32)
            return carry
        lax.fori_loop(0, CHUNK, zrow, 0)
        for j in range(ROWS_PT // CHUNK):
            pltpu.sync_copy(
                hrows, acc.at[pl.ds(sid * ROWS_PT + j * CHUNK, CHUNK)])

        pltpu.sync_copy(dst_hbm.at[tid], dst_v)
        plsc.subcore_barrier()

        def chunk(g, carry):
            base = tid * EPT + g * CHUNK
            pltpu.sync_copy(src_hbm.at[pl.ds(base, CHUNK)], src_c)
            pltpu.sync_copy(ew_hbm.at[pl.ds(base, CHUNK)], ew_c)
            pltpu.async_copy(h_hbm.at[src_c], hrows, sem).wait()
            pltpu.sync_copy(emb_hbm.at[pl.ds(base, CHUNK)], embv)

            def edge_grp(eg, c2):
                wvec = ew_c[pl.ds(eg * 16, 16)]
                for e16 in range(16):
                    w = wvec[e16]
                    r = eg * 16 + e16
                    for k in range(d // 16):
                        s = pl.ds(k * 16, 16)
                        v = hrows[r, s] + embv[r, s]
                        hrows[r, s] = jnp.maximum(v, 0.0) * w
                return c2
            lax.fori_loop(0, CHUNK // 16, edge_grp, 0)

            pltpu.sync_copy(hrows, acc.at[dst_v.at[g]], add=True)
            return carry
        lax.fori_loop(0, NCHUNK, chunk, 0)

        plsc.subcore_barrier()
        r0 = sid * ROWS_PT
        pltpu.sync_copy(acc.at[pl.ds(r0, ROWS_PT)],
                        out_hbm.at[cid, pl.ds(r0, ROWS_PT)])

    return pl.kernel(
        body,
        out_type=jax.ShapeDtypeStruct((SC_CORES, N_PAD, d), jnp.float32),
        mesh=mesh,
        scratch_types=[
            pltpu.VMEM_SHARED((N_PAD, d), jnp.float32),
            pltpu.VMEM((NCHUNK, CHUNK), jnp.int32),
            pltpu.VMEM((CHUNK,), jnp.int32),
            pltpu.VMEM((CHUNK,), jnp.float32),
            pltpu.VMEM((CHUNK, d), jnp.float32),
            pltpu.VMEM((CHUNK, d), jnp.float32),
            pltpu.SemaphoreType.DMA,
        ],
    )


_mp128 = _make_mp(DIM)


def _edge_mlp_body(ea_ref, w1_ref, b1_ref, w2_ref, b2_ref, out_ref):
    ea = ea_ref[...]
    t = jax.lax.dot_general(ea, w1_ref[...], (((1,), (0,)), ((), ())),
                            preferred_element_type=jnp.float32)
    t = jnp.maximum(t + b1_ref[...], 0.0)
    o = jax.lax.dot_general(t, w2_ref[...], (((1,), (0,)), ((), ())),
                            preferred_element_type=jnp.float32)
    out_ref[...] = o + b2_ref[...]


def _edge_mlp(ea, w1, b1, w2, b2):
    dp = w1.shape[1]
    BE = 2000
    return pl.pallas_call(
        _edge_mlp_body,
        grid=(E // BE,),
        in_specs=[
            pl.BlockSpec((BE, 3), lambda i: (i, 0)),
            pl.BlockSpec((3, dp), lambda i: (0, 0)),
            pl.BlockSpec((1, dp), lambda i: (0, 0)),
            pl.BlockSpec((dp, dp), lambda i: (0, 0)),
            pl.BlockSpec((1, dp), lambda i: (0, 0)),
        ],
        out_specs=pl.BlockSpec((BE, dp), lambda i: (i, 0)),
        out_shape=jax.ShapeDtypeStruct((E, dp), jnp.float32),
        compiler_params=pltpu.CompilerParams(
            dimension_semantics=("parallel",)),
    )(ea, w1, b1, w2, b2)


BN_BLK = 1000


def _node_mlp_body(h_ref, a0_ref, a1_ref, eps_ref, w1_ref, b1_ref,
                   w2_ref, b2_ref, y_ref, st_ref):
    i = pl.program_id(0)
    u = h_ref[...] * (1.0 + eps_ref[0, 0]) + a0_ref[0] + a1_ref[0]
    t = jax.lax.dot_general(u, w1_ref[...], (((1,), (0,)), ((), ())),
                            preferred_element_type=jnp.float32)
    t = jnp.maximum(t + b1_ref[...], 0.0)
    y = jax.lax.dot_general(t, w2_ref[...], (((1,), (0,)), ((), ())),
                            preferred_element_type=jnp.float32) + b2_ref[...]
    y_ref[...] = y

    @pl.when(i == 0)
    def _():
        st_ref[...] = jnp.zeros_like(st_ref)
    st_ref[0:1, :] += jnp.sum(y, axis=0, keepdims=True)
    st_ref[1:2, :] += jnp.sum(y * y, axis=0, keepdims=True)


def _node_mlp(h, agg2, eps, w1, b1, w2, b2):
    dp = h.shape[1]
    return pl.pallas_call(
        _node_mlp_body,
        grid=(N // BN_BLK,),
        in_specs=[
            pl.BlockSpec((BN_BLK, dp), lambda i: (i, 0)),
            pl.BlockSpec((1, BN_BLK, dp), lambda i: (0, i, 0)),
            pl.BlockSpec((1, BN_BLK, dp), lambda i: (1, i, 0)),
            pl.BlockSpec((1, 1), lambda i: (0, 0)),
            pl.BlockSpec((dp, dp), lambda i: (0, 0)),
            pl.BlockSpec((1, dp), lambda i: (0, 0)),
            pl.BlockSpec((dp, DIM), lambda i: (0, 0)),
            pl.BlockSpec((1, DIM), lambda i: (0, 0)),
        ],
        out_specs=[
            pl.BlockSpec((BN_BLK, DIM), lambda i: (i, 0)),
            pl.BlockSpec((2, DIM), lambda i: (0, 0)),
        ],
        out_shape=[
            jax.ShapeDtypeStruct((N, DIM), jnp.float32),
            jax.ShapeDtypeStruct((2, DIM), jnp.float32),
        ],
        compiler_params=pltpu.CompilerParams(
            dimension_semantics=("arbitrary",)),
    )(h, agg2, agg2, eps, w1, b1, w2, b2)


def _bn_pool_body(y_ref, st_ref, g_ref, b_ref, bt_ref, h_ref, pool_ref,
                  cnt_ref):
    i = pl.program_id(0)
    y = y_ref[...]
    mu = st_ref[0:1, :] / N
    var = st_ref[1:2, :] / N - mu * mu
    xn = g_ref[...] * (y - mu) * jax.lax.rsqrt(var + 1e-5) + b_ref[...]
    h = jnp.maximum(xn, 0.0)
    h_ref[...] = h

    seg = bt_ref[0, 0, :]
    onehot = (seg[None, :] ==
              jax.lax.broadcasted_iota(jnp.int32, (NG, BN_BLK), 0)
              ).astype(jnp.float32)

    @pl.when(i == 0)
    def _():
        pool_ref[...] = jnp.zeros_like(pool_ref)
        cnt_ref[...] = jnp.zeros_like(cnt_ref)
    pool_ref[...] += jax.lax.dot_general(
        onehot, h, (((1,), (0,)), ((), ())),
        preferred_element_type=jnp.float32)
    cnt_ref[...] += jax.lax.dot_general(
        onehot, jnp.ones_like(h), (((1,), (0,)), ((), ())),
        preferred_element_type=jnp.float32)


def _bn_pool(y, st, g, b, batch3):
    return pl.pallas_call(
        _bn_pool_body,
        grid=(N // BN_BLK,),
        in_specs=[
            pl.BlockSpec((BN_BLK, DIM), lambda i: (i, 0)),
            pl.BlockSpec((2, DIM), lambda i: (0, 0)),
            pl.BlockSpec((1, DIM), lambda i: (0, 0)),
            pl.BlockSpec((1, DIM), lambda i: (0, 0)),
            pl.BlockSpec((1, 1, BN_BLK), lambda i: (i, 0, 0)),
        ],
        out_specs=[
            pl.BlockSpec((BN_BLK, DIM), lambda i: (i, 0)),
            pl.BlockSpec((NG, DIM), lambda i: (0, 0)),
            pl.BlockSpec((NG, DIM), lambda i: (0, 0)),
        ],
        out_shape=[
            jax.ShapeDtypeStruct((N, DIM), jnp.float32),
            jax.ShapeDtypeStruct((NG, DIM), jnp.float32),
            jax.ShapeDtypeStruct((NG, DIM), jnp.float32),
        ],
        compiler_params=pltpu.CompilerParams(
            dimension_semantics=("arbitrary",)),
    )(y, st, g, b, batch3)


def _bn_rows(t, g, b):
    mu = jnp.mean(t, axis=0, keepdims=True)
    var = jnp.mean(t * t, axis=0, keepdims=True) - mu * mu
    return g * (t - mu) * jax.lax.rsqrt(var + 1e-5) + b


def _tail_body(p0_ref, p1_ref, p2_ref, cnt_ref, w1_ref, b1_ref, g1_ref,
               bb1_ref, w2_ref, b2_ref, g2_ref, bb2_ref, w3_ref, b3_ref,
               ig_ref, out_ref):
    c = jnp.maximum(cnt_ref[...], 1.0)
    g = jnp.concatenate(
        [p0_ref[...] / c, p1_ref[...] / c, p2_ref[...] / c], axis=1)
    t = jax.lax.dot_general(g, w1_ref[...], (((1,), (0,)), ((), ())),
                            preferred_element_type=jnp.float32) + b1_ref[...]
    t = jnp.maximum(_bn_rows(t, g1_ref[...], bb1_ref[...]), 0.0)
    t = jax.lax.dot_general(t, w2_ref[...], (((1,), (0,)), ((), ())),
                            preferred_element_type=jnp.float32) + b2_ref[...]
    t = jnp.maximum(_bn_rows(t, g2_ref[...], bb2_ref[...]), 0.0)
    ig = ig_ref[0, 0, :]
    oh = (ig[None, :] ==
          jax.lax.broadcasted_iota(jnp.int32, (NS_GRAPH, NG), 0)
          ).astype(jnp.float32)
    ssum = jax.lax.dot_general(oh, t, (((1,), (0,)), ((), ())),
                               preferred_element_type=jnp.float32)
    scnt = jnp.maximum(
        jax.lax.dot_general(oh, jnp.ones_like(t), (((1,), (0,)), ((), ())),
                            preferred_element_type=jnp.float32), 1.0)
    s = ssum / scnt
    out_ref[...] = jax.lax.dot_general(
        s, w3_ref[...], (((1,), (0,)), ((), ())),
        preferred_element_type=jnp.float32) + b3_ref[...]


def _tail(p0, p1, p2, cnt, w1, b1, g1, bb1, w2, b2, g2, bb2, w3, b3, ig3):
    full = lambda shp: pl.BlockSpec(shp, lambda: tuple(0 for _ in shp))
    return pl.pallas_call(
        _tail_body,
        in_specs=[
            full((NG, DIM)), full((NG, DIM)), full((NG, DIM)),
            full((NG, DIM)),
            full((3 * DIM, DIM)), full((1, DIM)), full((1, DIM)),
            full((1, DIM)),
            full((DIM, DIM)), full((1, DIM)), full((1, DIM)), full((1, DIM)),
            full((DIM, 1)), full((1, 1)),
            full((1, 1, NG)),
        ],
        out_specs=full((NS_GRAPH, 1)),
        out_shape=jax.ShapeDtypeStruct((NS_GRAPH, 1), jnp.float32),
    )(p0, p1, p2, cnt, w1, b1, g1, bb1, w2, b2, g2, bb2, w3, b3, ig3)


def _pad2(a, rows, cols):
    return jnp.pad(a, ((0, rows - a.shape[0]), (0, cols - a.shape[1])))


def kernel(x, edge_attr, edge_weight, params, edge_index, batch,
           inter_graph_idx):
    ei = edge_index.astype(jnp.int32)
    src3 = ei[0]
    dst3 = ei[1].reshape(NTILES, NCHUNK, CHUNK)
    ew3 = edge_weight
    batch3 = batch.astype(jnp.int32).reshape(N // BN_BLK, 1, BN_BLK)
    ig3 = inter_graph_idx.astype(jnp.int32).reshape(1, 1, NG)

    h = jnp.pad(x, ((0, 0), (0, DIM - x.shape[1])))

    pools = []
    cnt = None
    for i in range(3):
        cp = params["convs"][i]
        dp = DIM

        w_e1 = _pad2(cp["be1"]["W"], 3, dp)
        b_e1 = _pad2(cp["be1"]["b"][None, :], 1, dp)
        w_e2 = _pad2(cp["be2"]["W"], dp, dp)
        b_e2 = _pad2(cp["be2"]["b"][None, :], 1, dp)
        emb = _edge_mlp(edge_attr, w_e1, b_e1, w_e2, b_e2)

        agg2 = _mp128(h, emb, src3, dst3, ew3)

        w_m1 = _pad2(cp["m1"]["W"], dp, dp)
        b_m1 = _pad2(cp["m1"]["b"][None, :], 1, dp)
        w_m2 = _pad2(cp["m2"]["W"], dp, DIM)
        b_m2 = cp["m2"]["b"][None, :]
        eps = cp["eps"].reshape(1, 1)
        y, st = _node_mlp(h, agg2, eps, w_m1, b_m1, w_m2, b_m2)

        h, pool, cnt_i = _bn_pool(y, st, params["bn_g"][i][None, :],
                                  params["bn_b"][i][None, :], batch3)
        pools.append(pool)
        if cnt is None:
            cnt = cnt_i

    out = _tail(pools[0], pools[1], pools[2], cnt,
                params["fc1"]["W"], params["fc1"]["b"][None, :],
                params["bn1_g"][None, :], params["bn1_b"][None, :],
                params["fc2"]["W"], params["fc2"]["b"][None, :],
                params["bn2_g"][None, :], params["bn2_b"][None, :],
                params["fc3"]["W"], params["fc3"]["b"][None, :],
                ig3)
    return out.reshape(-1)


# speedup vs baseline: 4.7671x; 2.2965x over previous
import jax
import jax.numpy as jnp
from jax import lax
from jax.experimental import pallas as pl
from jax.experimental.pallas import tpu as pltpu
from jax.experimental.pallas import tpu_sc as plsc

N = 10000
E = 320000
DIM = 128
NG = 64
NS_GRAPH = 8

SC_CORES = 2
SC_SUBCORES = 16
NTILES = SC_CORES * SC_SUBCORES
CHUNK = 64
NC_TOT = E // CHUNK
NC_BASE = NC_TOT // NTILES
NC_REM = NC_TOT % NTILES
N_PAD = 10240
ROWS_PT = N_PAD // SC_SUBCORES


def _make_mp(d):
    mesh = plsc.VectorSubcoreMesh(core_axis_name="c", subcore_axis_name="s")

    def body(h_hbm, emb_hbm, src_hbm, dst_hbm, ew_hbm, out_hbm,
             acc, src_c, dst_c, ew_c, hrows, embv,
             sem_src, sem_dst, sem_ew, sem_g, sem_e, sem_sc):
        cid = lax.axis_index("c")
        sid = lax.axis_index("s")
        tid = cid * SC_SUBCORES + sid
        ncnk = jnp.where(tid < NC_REM, NC_BASE + 1, NC_BASE)

        def zrow(i, carry):
            for k in range(d // 16):
                hrows[0, i, pl.ds(k * 16, 16)] = jnp.zeros((16,), jnp.float32)
            return carry
        lax.fori_loop(0, CHUNK, zrow, 0)
        for j in range(ROWS_PT // CHUNK):
            pltpu.sync_copy(
                hrows.at[0],
                acc.at[pl.ds(sid * ROWS_PT + j * CHUNK, CHUNK)])
        plsc.subcore_barrier()

        def ebase(g):
            return (tid + g * NTILES) * CHUNK

        def issue_sde(g):
            s4 = jnp.bitwise_and(g, 3)
            base = ebase(g)
            pltpu.async_copy(src_hbm.at[pl.ds(base, CHUNK)],
                             src_c.at[s4], sem_src.at[s4])
            pltpu.async_copy(dst_hbm.at[pl.ds(base, CHUNK)],
                             dst_c.at[s4], sem_dst.at[s4])
            pltpu.async_copy(ew_hbm.at[pl.ds(base, CHUNK)],
                             ew_c.at[s4], sem_ew.at[s4])

        def wait_src(g):
            s4 = jnp.bitwise_and(g, 3)
            base = ebase(g)
            pltpu.make_async_copy(src_hbm.at[pl.ds(base, CHUNK)],
                                  src_c.at[s4], sem_src.at[s4]).wait()

        def wait_de(g):
            s4 = jnp.bitwise_and(g, 3)
            base = ebase(g)
            pltpu.make_async_copy(dst_hbm.at[pl.ds(base, CHUNK)],
                                  dst_c.at[s4], sem_dst.at[s4]).wait()
            pltpu.make_async_copy(ew_hbm.at[pl.ds(base, CHUNK)],
                                  ew_c.at[s4], sem_ew.at[s4]).wait()

        def issue_rows(g):
            s4 = jnp.bitwise_and(g, 3)
            p2 = jnp.bitwise_and(g, 1)
            base = ebase(g)
            pltpu.async_copy(h_hbm.at[src_c.at[s4]], hrows.at[p2],
                             sem_g.at[p2])
            pltpu.async_copy(emb_hbm.at[pl.ds(base, CHUNK)], embv.at[p2],
                             sem_e.at[p2])

        def wait_rows(g):
            s4 = jnp.bitwise_and(g, 3)
            p2 = jnp.bitwise_and(g, 1)
            base = ebase(g)
            pltpu.make_async_copy(h_hbm.at[src_c.at[s4]], hrows.at[p2],
                                  sem_g.at[p2]).wait()
            pltpu.make_async_copy(emb_hbm.at[pl.ds(base, CHUNK)],
                                  embv.at[p2], sem_e.at[p2]).wait()

        def issue_scatter(g):
            s4 = jnp.bitwise_and(g, 3)
            p2 = jnp.bitwise_and(g, 1)
            pltpu.async_copy(hrows.at[p2], acc.at[dst_c.at[s4]],
                             sem_sc.at[p2], add=True)

        def wait_scatter(g):
            s4 = jnp.bitwise_and(g, 3)
            p2 = jnp.bitwise_and(g, 1)
            pltpu.make_async_copy(hrows.at[p2], acc.at[dst_c.at[s4]],
                                  sem_sc.at[p2]).wait()

        def compute(g):
            p2 = jnp.bitwise_and(g, 1)
            s4 = jnp.bitwise_and(g, 3)

            def edge_grp(eg, c2):
                wvec = ew_c[s4, pl.ds(eg * 16, 16)]
                for e16 in range(16):
                    w = wvec[e16]
                    r = eg * 16 + e16
                    for k in range(d // 16):
                        s = pl.ds(k * 16, 16)
                        hrows[p2, r, s] = (
                            jnp.maximum(hrows[p2, r, s] + embv[p2, r, s],
                                        0.0) * w)
                return c2
            lax.fori_loop(0, CHUNK // 16, edge_grp, 0)

        issue_sde(0)
        issue_sde(1)
        wait_src(0)
        issue_rows(0)

        def chunk_iter(g, carry):
            @pl.when(g < ncnk)
            def _():
                @pl.when(g + 1 < ncnk)
                def _():
                    wait_src(g + 1)

                    @pl.when(g >= 1)
                    def _():
                        wait_scatter(g - 1)
                    issue_rows(g + 1)
                wait_rows(g)
                wait_de(g)

                @pl.when(g + 2 < ncnk)
                def _():
                    issue_sde(g + 2)
                compute(g)
                issue_scatter(g)
            return carry
        lax.fori_loop(0, NC_BASE + 1, chunk_iter, 0, unroll=False)

        wait_scatter(ncnk - 2)
        wait_scatter(ncnk - 1)

        plsc.subcore_barrier()
        r0 = sid * ROWS_PT
        pltpu.sync_copy(acc.at[pl.ds(r0, ROWS_PT)],
                        out_hbm.at[cid, pl.ds(r0, ROWS_PT)])

    return pl.kernel(
        body,
        out_type=jax.ShapeDtypeStruct((SC_CORES, N_PAD, d), jnp.float32),
        mesh=mesh,
        compiler_params=pltpu.CompilerParams(needs_layout_passes=False),
        scratch_types=[
            pltpu.VMEM_SHARED((N_PAD, d), jnp.float32),
            pltpu.VMEM((4, CHUNK), jnp.int32),
            pltpu.VMEM((4, CHUNK), jnp.int32),
            pltpu.VMEM((4, CHUNK), jnp.float32),
            pltpu.VMEM((2, CHUNK, d), jnp.float32),
            pltpu.VMEM((2, CHUNK, d), jnp.float32),
            pltpu.SemaphoreType.DMA((4,)),
            pltpu.SemaphoreType.DMA((4,)),
            pltpu.SemaphoreType.DMA((4,)),
            pltpu.SemaphoreType.DMA((2,)),
            pltpu.SemaphoreType.DMA((2,)),
            pltpu.SemaphoreType.DMA((2,)),
        ],
    )


_mp128 = _make_mp(DIM)


def _edge_mlp_body(ea_ref, w1_ref, b1_ref, w2_ref, b2_ref, out_ref):
    ea = ea_ref[...]
    t = jax.lax.dot_general(ea, w1_ref[...], (((1,), (0,)), ((), ())),
                            preferred_element_type=jnp.float32)
    t = jnp.maximum(t + b1_ref[...], 0.0)
    o = jax.lax.dot_general(t, w2_ref[...], (((1,), (0,)), ((), ())),
                            preferred_element_type=jnp.float32)
    out_ref[...] = o + b2_ref[...]


def _edge_mlp(ea, w1, b1, w2, b2):
    dp = w1.shape[1]
    BE = 2000
    return pl.pallas_call(
        _edge_mlp_body,
        grid=(E // BE,),
        in_specs=[
            pl.BlockSpec((BE, 3), lambda i: (i, 0)),
            pl.BlockSpec((3, dp), lambda i: (0, 0)),
            pl.BlockSpec((1, dp), lambda i: (0, 0)),
            pl.BlockSpec((dp, dp), lambda i: (0, 0)),
            pl.BlockSpec((1, dp), lambda i: (0, 0)),
        ],
        out_specs=pl.BlockSpec((BE, dp), lambda i: (i, 0)),
        out_shape=jax.ShapeDtypeStruct((E, dp), jnp.float32),
        compiler_params=pltpu.CompilerParams(
            dimension_semantics=("parallel",)),
    )(ea, w1, b1, w2, b2)


BN_BLK = 1000


def _node_mlp_body(h_ref, a0_ref, a1_ref, eps_ref, w1_ref, b1_ref,
                   w2_ref, b2_ref, y_ref, st_ref):
    i = pl.program_id(0)
    u = h_ref[...] * (1.0 + eps_ref[0, 0]) + a0_ref[0] + a1_ref[0]
    t = jax.lax.dot_general(u, w1_ref[...], (((1,), (0,)), ((), ())),
                            preferred_element_type=jnp.float32)
    t = jnp.maximum(t + b1_ref[...], 0.0)
    y = jax.lax.dot_general(t, w2_ref[...], (((1,), (0,)), ((), ())),
                            preferred_element_type=jnp.float32) + b2_ref[...]
    y_ref[...] = y

    @pl.when(i == 0)
    def _():
        st_ref[...] = jnp.zeros_like(st_ref)
    st_ref[0:1, :] += jnp.sum(y, axis=0, keepdims=True)
    st_ref[1:2, :] += jnp.sum(y * y, axis=0, keepdims=True)


def _node_mlp(h, agg2, eps, w1, b1, w2, b2):
    dp = h.shape[1]
    return pl.pallas_call(
        _node_mlp_body,
        grid=(N // BN_BLK,),
        in_specs=[
            pl.BlockSpec((BN_BLK, dp), lambda i: (i, 0)),
            pl.BlockSpec((1, BN_BLK, dp), lambda i: (0, i, 0)),
            pl.BlockSpec((1, BN_BLK, dp), lambda i: (1, i, 0)),
            pl.BlockSpec((1, 1), lambda i: (0, 0)),
            pl.BlockSpec((dp, dp), lambda i: (0, 0)),
            pl.BlockSpec((1, dp), lambda i: (0, 0)),
            pl.BlockSpec((dp, DIM), lambda i: (0, 0)),
            pl.BlockSpec((1, DIM), lambda i: (0, 0)),
        ],
        out_specs=[
            pl.BlockSpec((BN_BLK, DIM), lambda i: (i, 0)),
            pl.BlockSpec((2, DIM), lambda i: (0, 0)),
        ],
        out_shape=[
            jax.ShapeDtypeStruct((N, DIM), jnp.float32),
            jax.ShapeDtypeStruct((2, DIM), jnp.float32),
        ],
        compiler_params=pltpu.CompilerParams(
            dimension_semantics=("arbitrary",)),
    )(h, agg2, agg2, eps, w1, b1, w2, b2)


def _bn_pool_body(y_ref, st_ref, g_ref, b_ref, bt_ref, h_ref, pool_ref,
                  cnt_ref):
    i = pl.program_id(0)
    y = y_ref[...]
    mu = st_ref[0:1, :] / N
    var = st_ref[1:2, :] / N - mu * mu
    xn = g_ref[...] * (y - mu) * jax.lax.rsqrt(var + 1e-5) + b_ref[...]
    h = jnp.maximum(xn, 0.0)
    h_ref[...] = h

    seg = bt_ref[0, 0, :]
    onehot = (seg[None, :] ==
              jax.lax.broadcasted_iota(jnp.int32, (NG, BN_BLK), 0)
              ).astype(jnp.float32)

    @pl.when(i == 0)
    def _():
        pool_ref[...] = jnp.zeros_like(pool_ref)
        cnt_ref[...] = jnp.zeros_like(cnt_ref)
    pool_ref[...] += jax.lax.dot_general(
        onehot, h, (((1,), (0,)), ((), ())),
        preferred_element_type=jnp.float32)
    cnt_ref[...] += jax.lax.dot_general(
        onehot, jnp.ones_like(h), (((1,), (0,)), ((), ())),
        preferred_element_type=jnp.float32)


def _bn_pool(y, st, g, b, batch3):
    return pl.pallas_call(
        _bn_pool_body,
        grid=(N // BN_BLK,),
        in_specs=[
            pl.BlockSpec((BN_BLK, DIM), lambda i: (i, 0)),
            pl.BlockSpec((2, DIM), lambda i: (0, 0)),
            pl.BlockSpec((1, DIM), lambda i: (0, 0)),
            pl.BlockSpec((1, DIM), lambda i: (0, 0)),
            pl.BlockSpec((1, 1, BN_BLK), lambda i: (i, 0, 0)),
        ],
        out_specs=[
            pl.BlockSpec((BN_BLK, DIM), lambda i: (i, 0)),
            pl.BlockSpec((NG, DIM), lambda i: (0, 0)),
            pl.BlockSpec((NG, DIM), lambda i: (0, 0)),
        ],
        out_shape=[
            jax.ShapeDtypeStruct((N, DIM), jnp.float32),
            jax.ShapeDtypeStruct((NG, DIM), jnp.float32),
            jax.ShapeDtypeStruct((NG, DIM), jnp.float32),
        ],
        compiler_params=pltpu.CompilerParams(
            dimension_semantics=("arbitrary",)),
    )(y, st, g, b, batch3)


def _bn_rows(t, g, b):
    mu = jnp.mean(t, axis=0, keepdims=True)
    var = jnp.mean(t * t, axis=0, keepdims=True) - mu * mu
    return g * (t - mu) * jax.lax.rsqrt(var + 1e-5) + b


def _tail_body(p0_ref, p1_ref, p2_ref, cnt_ref, w1_ref, b1_ref, g1_ref,
               bb1_ref, w2_ref, b2_ref, g2_ref, bb2_ref, w3_ref, b3_ref,
               ig_ref, out_ref):
    c = jnp.maximum(cnt_ref[...], 1.0)
    g = jnp.concatenate(
        [p0_ref[...] / c, p1_ref[...] / c, p2_ref[...] / c], axis=1)
    t = jax.lax.dot_general(g, w1_ref[...], (((1,), (0,)), ((), ())),
                            preferred_element_type=jnp.float32) + b1_ref[...]
    t = jnp.maximum(_bn_rows(t, g1_ref[...], bb1_ref[...]), 0.0)
    t = jax.lax.dot_general(t, w2_ref[...], (((1,), (0,)), ((), ())),
                            preferred_element_type=jnp.float32) + b2_ref[...]
    t = jnp.maximum(_bn_rows(t, g2_ref[...], bb2_ref[...]), 0.0)
    ig = ig_ref[0, 0, :]
    oh = (ig[None, :] ==
          jax.lax.broadcasted_iota(jnp.int32, (NS_GRAPH, NG), 0)
          ).astype(jnp.float32)
    ssum = jax.lax.dot_general(oh, t, (((1,), (0,)), ((), ())),
                               preferred_element_type=jnp.float32)
    scnt = jnp.maximum(
        jax.lax.dot_general(oh, jnp.ones_like(t), (((1,), (0,)), ((), ())),
                            preferred_element_type=jnp.float32), 1.0)
    s = ssum / scnt
    out_ref[...] = jax.lax.dot_general(
        s, w3_ref[...], (((1,), (0,)), ((), ())),
        preferred_element_type=jnp.float32) + b3_ref[...]


def _tail(p0, p1, p2, cnt, w1, b1, g1, bb1, w2, b2, g2, bb2, w3, b3, ig3):
    full = lambda shp: pl.BlockSpec(shp, lambda: tuple(0 for _ in shp))
    return pl.pallas_call(
        _tail_body,
        in_specs=[
            full((NG, DIM)), full((NG, DIM)), full((NG, DIM)),
            full((NG, DIM)),
            full((3 * DIM, DIM)), full((1, DIM)), full((1, DIM)),
            full((1, DIM)),
            full((DIM, DIM)), full((1, DIM)), full((1, DIM)), full((1, DIM)),
            full((DIM, 1)), full((1, 1)),
            full((1, 1, NG)),
        ],
        out_specs=full((NS_GRAPH, 1)),
        out_shape=jax.ShapeDtypeStruct((NS_GRAPH, 1), jnp.float32),
    )(p0, p1, p2, cnt, w1, b1, g1, bb1, w2, b2, g2, bb2, w3, b3, ig3)


def _pad2(a, rows, cols):
    return jnp.pad(a, ((0, rows - a.shape[0]), (0, cols - a.shape[1])))


def kernel(x, edge_attr, edge_weight, params, edge_index, batch,
           inter_graph_idx):
    ei = edge_index.astype(jnp.int32)
    src1 = ei[0]
    dst1 = ei[1]
    ew1 = edge_weight
    batch3 = batch.astype(jnp.int32).reshape(N // BN_BLK, 1, BN_BLK)
    ig3 = inter_graph_idx.astype(jnp.int32).reshape(1, 1, NG)

    h = jnp.pad(x, ((0, 0), (0, DIM - x.shape[1])))

    pools = []
    cnt = None
    for i in range(3):
        cp = params["convs"][i]
        dp = DIM

        w_e1 = _pad2(cp["be1"]["W"], 3, dp)
        b_e1 = _pad2(cp["be1"]["b"][None, :], 1, dp)
        w_e2 = _pad2(cp["be2"]["W"], dp, dp)
        b_e2 = _pad2(cp["be2"]["b"][None, :], 1, dp)
        emb = _edge_mlp(edge_attr, w_e1, b_e1, w_e2, b_e2)

        agg2 = _mp128(h, emb, src1, dst1, ew1)

        w_m1 = _pad2(cp["m1"]["W"], dp, dp)
        b_m1 = _pad2(cp["m1"]["b"][None, :], 1, dp)
        w_m2 = _pad2(cp["m2"]["W"], dp, DIM)
        b_m2 = cp["m2"]["b"][None, :]
        eps = cp["eps"].reshape(1, 1)
        y, st = _node_mlp(h, agg2, eps, w_m1, b_m1, w_m2, b_m2)

        h, pool, cnt_i = _bn_pool(y, st, params["bn_g"][i][None, :],
                                  params["bn_b"][i][None, :], batch3)
        pools.append(pool)
        if cnt is None:
            cnt = cnt_i

    out = _tail(pools[0], pools[1], pools[2], cnt,
                params["fc1"]["W"], params["fc1"]["b"][None, :],
                params["bn1_g"][None, :], params["bn1_b"][None, :],
                params["fc2"]["W"], params["fc2"]["b"][None, :],
                params["bn2_g"][None, :], params["bn2_b"][None, :],
                params["fc3"]["W"], params["fc3"]["b"][None, :],
                ig3)
    return out.reshape(-1)


# speedup vs baseline: 4.8498x; 1.0174x over previous
import jax
import jax.numpy as jnp
from jax import lax
from jax.experimental import pallas as pl
from jax.experimental.pallas import tpu as pltpu
from jax.experimental.pallas import tpu_sc as plsc

N = 10000
E = 320000
DIM = 128
NG = 64
NS_GRAPH = 8

SC_CORES = 2
SC_SUBCORES = 16
NTILES = SC_CORES * SC_SUBCORES
CHUNK = 64
NC_TOT = E // CHUNK
NC_BASE = NC_TOT // NTILES
NC_REM = NC_TOT % NTILES
N_PAD = 10240
ROWS_PT = N_PAD // SC_SUBCORES


def _make_mp(d):
    mesh = plsc.VectorSubcoreMesh(core_axis_name="c", subcore_axis_name="s")

    def body(h_hbm, emb_hbm, src_hbm, dst_hbm, ew_hbm, out_hbm,
             acc, src_c, dst_c, ew_c, hrows, embv,
             sem_src, sem_dst, sem_ew, sem_g, sem_e, sem_sc):
        cid = lax.axis_index("c")
        sid = lax.axis_index("s")
        tid = cid * SC_SUBCORES + sid
        ncnk = jnp.where(tid < NC_REM, NC_BASE + 1, NC_BASE)

        def zrow(i, carry):
            for k in range(d // 16):
                hrows[0, i, pl.ds(k * 16, 16)] = jnp.zeros((16,), jnp.float32)
            return carry
        lax.fori_loop(0, CHUNK, zrow, 0)
        for j in range(ROWS_PT // CHUNK):
            pltpu.sync_copy(
                hrows.at[0],
                acc.at[pl.ds(sid * ROWS_PT + j * CHUNK, CHUNK)])
        plsc.subcore_barrier()

        def ebase(g):
            return (tid + g * NTILES) * CHUNK

        def issue_sde(g):
            s4 = jnp.bitwise_and(g, 3)
            base = ebase(g)
            pltpu.async_copy(src_hbm.at[pl.ds(base, CHUNK)],
                             src_c.at[s4], sem_src.at[s4])
            pltpu.async_copy(dst_hbm.at[pl.ds(base, CHUNK)],
                             dst_c.at[s4], sem_dst.at[s4])
            pltpu.async_copy(ew_hbm.at[pl.ds(base, CHUNK)],
                             ew_c.at[s4], sem_ew.at[s4])

        def wait_src(g):
            s4 = jnp.bitwise_and(g, 3)
            base = ebase(g)
            pltpu.make_async_copy(src_hbm.at[pl.ds(base, CHUNK)],
                                  src_c.at[s4], sem_src.at[s4]).wait()

        def wait_de(g):
            s4 = jnp.bitwise_and(g, 3)
            base = ebase(g)
            pltpu.make_async_copy(dst_hbm.at[pl.ds(base, CHUNK)],
                                  dst_c.at[s4], sem_dst.at[s4]).wait()
            pltpu.make_async_copy(ew_hbm.at[pl.ds(base, CHUNK)],
                                  ew_c.at[s4], sem_ew.at[s4]).wait()

        def issue_rows(g):
            s4 = jnp.bitwise_and(g, 3)
            p2 = jnp.bitwise_and(g, 1)
            base = ebase(g)
            pltpu.async_copy(h_hbm.at[src_c.at[s4]], hrows.at[p2],
                             sem_g.at[p2])
            pltpu.async_copy(emb_hbm.at[pl.ds(base, CHUNK)], embv.at[p2],
                             sem_e.at[p2])

        def wait_rows(g):
            s4 = jnp.bitwise_and(g, 3)
            p2 = jnp.bitwise_and(g, 1)
            base = ebase(g)
            pltpu.make_async_copy(h_hbm.at[src_c.at[s4]], hrows.at[p2],
                                  sem_g.at[p2]).wait()
            pltpu.make_async_copy(emb_hbm.at[pl.ds(base, CHUNK)],
                                  embv.at[p2], sem_e.at[p2]).wait()

        def issue_scatter(g):
            s4 = jnp.bitwise_and(g, 3)
            p2 = jnp.bitwise_and(g, 1)
            pltpu.async_copy(hrows.at[p2], acc.at[dst_c.at[s4]],
                             sem_sc.at[p2], add=True)

        def wait_scatter(g):
            s4 = jnp.bitwise_and(g, 3)
            p2 = jnp.bitwise_and(g, 1)
            pltpu.make_async_copy(hrows.at[p2], acc.at[dst_c.at[s4]],
                                  sem_sc.at[p2]).wait()

        def compute(g):
            p2 = jnp.bitwise_and(g, 1)
            s4 = jnp.bitwise_and(g, 3)

            def edge_grp(eg, c2):
                wvec = ew_c[s4, pl.ds(eg * 16, 16)]
                for e16 in range(16):
                    w = wvec[e16]
                    r = eg * 16 + e16
                    for k in range(d // 16):
                        s = pl.ds(k * 16, 16)
                        hrows[p2, r, s] = (
                            jnp.maximum(hrows[p2, r, s] + embv[p2, r, s],
                                        0.0) * w)
                return c2
            lax.fori_loop(0, CHUNK // 16, edge_grp, 0, unroll=True)

        issue_sde(0)
        issue_sde(1)
        wait_src(0)
        issue_rows(0)

        def chunk_iter(g, carry):
            @pl.when(g < ncnk)
            def _():
                @pl.when(g + 1 < ncnk)
                def _():
                    wait_src(g + 1)

                    @pl.when(g >= 1)
                    def _():
                        wait_scatter(g - 1)
                    issue_rows(g + 1)
                wait_rows(g)
                wait_de(g)

                @pl.when(g + 2 < ncnk)
                def _():
                    issue_sde(g + 2)
                compute(g)
                issue_scatter(g)
            return carry
        lax.fori_loop(0, NC_BASE + 1, chunk_iter, 0, unroll=False)

        wait_scatter(ncnk - 2)
        wait_scatter(ncnk - 1)

        plsc.subcore_barrier()
        r0 = sid * ROWS_PT
        pltpu.sync_copy(acc.at[pl.ds(r0, ROWS_PT)],
                        out_hbm.at[cid, pl.ds(r0, ROWS_PT)])

    return pl.kernel(
        body,
        out_type=jax.ShapeDtypeStruct((SC_CORES, N_PAD, d), jnp.float32),
        mesh=mesh,
        compiler_params=pltpu.CompilerParams(needs_layout_passes=False),
        scratch_types=[
            pltpu.VMEM_SHARED((N_PAD, d), jnp.float32),
            pltpu.VMEM((4, CHUNK), jnp.int32),
            pltpu.VMEM((4, CHUNK), jnp.int32),
            pltpu.VMEM((4, CHUNK), jnp.float32),
            pltpu.VMEM((2, CHUNK, d), jnp.float32),
            pltpu.VMEM((2, CHUNK, d), jnp.float32),
            pltpu.SemaphoreType.DMA((4,)),
            pltpu.SemaphoreType.DMA((4,)),
            pltpu.SemaphoreType.DMA((4,)),
            pltpu.SemaphoreType.DMA((2,)),
            pltpu.SemaphoreType.DMA((2,)),
            pltpu.SemaphoreType.DMA((2,)),
        ],
    )


_mp128 = _make_mp(DIM)


def _edge_mlp_body(ea_ref, w1_ref, b1_ref, w2_ref, b2_ref, out_ref):
    ea = ea_ref[...]
    t = jax.lax.dot_general(ea, w1_ref[...], (((1,), (0,)), ((), ())),
                            preferred_element_type=jnp.float32)
    t = jnp.maximum(t + b1_ref[...], 0.0)
    o = jax.lax.dot_general(t, w2_ref[...], (((1,), (0,)), ((), ())),
                            preferred_element_type=jnp.float32)
    out_ref[...] = o + b2_ref[...]


def _edge_mlp(ea, w1, b1, w2, b2):
    dp = w1.shape[1]
    BE = 2000
    return pl.pallas_call(
        _edge_mlp_body,
        grid=(E // BE,),
        in_specs=[
            pl.BlockSpec((BE, 3), lambda i: (i, 0)),
            pl.BlockSpec((3, dp), lambda i: (0, 0)),
            pl.BlockSpec((1, dp), lambda i: (0, 0)),
            pl.BlockSpec((dp, dp), lambda i: (0, 0)),
            pl.BlockSpec((1, dp), lambda i: (0, 0)),
        ],
        out_specs=pl.BlockSpec((BE, dp), lambda i: (i, 0)),
        out_shape=jax.ShapeDtypeStruct((E, dp), jnp.float32),
        compiler_params=pltpu.CompilerParams(
            dimension_semantics=("parallel",)),
    )(ea, w1, b1, w2, b2)


BN_BLK = 1000


def _node_mlp_body(h_ref, a0_ref, a1_ref, eps_ref, w1_ref, b1_ref,
                   w2_ref, b2_ref, y_ref, st_ref):
    i = pl.program_id(0)
    u = h_ref[...] * (1.0 + eps_ref[0, 0]) + a0_ref[0] + a1_ref[0]
    t = jax.lax.dot_general(u, w1_ref[...], (((1,), (0,)), ((), ())),
                            preferred_element_type=jnp.float32)
    t = jnp.maximum(t + b1_ref[...], 0.0)
    y = jax.lax.dot_general(t, w2_ref[...], (((1,), (0,)), ((), ())),
                            preferred_element_type=jnp.float32) + b2_ref[...]
    y_ref[...] = y

    @pl.when(i == 0)
    def _():
        st_ref[...] = jnp.zeros_like(st_ref)
    st_ref[0:1, :] += jnp.sum(y, axis=0, keepdims=True)
    st_ref[1:2, :] += jnp.sum(y * y, axis=0, keepdims=True)


def _node_mlp(h, agg2, eps, w1, b1, w2, b2):
    dp = h.shape[1]
    return pl.pallas_call(
        _node_mlp_body,
        grid=(N // BN_BLK,),
        in_specs=[
            pl.BlockSpec((BN_BLK, dp), lambda i: (i, 0)),
            pl.BlockSpec((1, BN_BLK, dp), lambda i: (0, i, 0)),
            pl.BlockSpec((1, BN_BLK, dp), lambda i: (1, i, 0)),
            pl.BlockSpec((1, 1), lambda i: (0, 0)),
            pl.BlockSpec((dp, dp), lambda i: (0, 0)),
            pl.BlockSpec((1, dp), lambda i: (0, 0)),
            pl.BlockSpec((dp, DIM), lambda i: (0, 0)),
            pl.BlockSpec((1, DIM), lambda i: (0, 0)),
        ],
        out_specs=[
            pl.BlockSpec((BN_BLK, DIM), lambda i: (i, 0)),
            pl.BlockSpec((2, DIM), lambda i: (0, 0)),
        ],
        out_shape=[
            jax.ShapeDtypeStruct((N, DIM), jnp.float32),
            jax.ShapeDtypeStruct((2, DIM), jnp.float32),
        ],
        compiler_params=pltpu.CompilerParams(
            dimension_semantics=("arbitrary",)),
    )(h, agg2, agg2, eps, w1, b1, w2, b2)


def _bn_pool_body(y_ref, st_ref, g_ref, b_ref, bt_ref, h_ref, pool_ref,
                  cnt_ref):
    i = pl.program_id(0)
    y = y_ref[...]
    mu = st_ref[0:1, :] / N
    var = st_ref[1:2, :] / N - mu * mu
    xn = g_ref[...] * (y - mu) * jax.lax.rsqrt(var + 1e-5) + b_ref[...]
    h = jnp.maximum(xn, 0.0)
    h_ref[...] = h

    seg = bt_ref[0, 0, :]
    onehot = (seg[None, :] ==
              jax.lax.broadcasted_iota(jnp.int32, (NG, BN_BLK), 0)
              ).astype(jnp.float32)

    @pl.when(i == 0)
    def _():
        pool_ref[...] = jnp.zeros_like(pool_ref)
        cnt_ref[...] = jnp.zeros_like(cnt_ref)
    pool_ref[...] += jax.lax.dot_general(
        onehot, h, (((1,), (0,)), ((), ())),
        preferred_element_type=jnp.float32)
    cnt_ref[...] += jax.lax.dot_general(
        onehot, jnp.ones_like(h), (((1,), (0,)), ((), ())),
        preferred_element_type=jnp.float32)


def _bn_pool(y, st, g, b, batch3):
    return pl.pallas_call(
        _bn_pool_body,
        grid=(N // BN_BLK,),
        in_specs=[
            pl.BlockSpec((BN_BLK, DIM), lambda i: (i, 0)),
            pl.BlockSpec((2, DIM), lambda i: (0, 0)),
            pl.BlockSpec((1, DIM), lambda i: (0, 0)),
            pl.BlockSpec((1, DIM), lambda i: (0, 0)),
            pl.BlockSpec((1, 1, BN_BLK), lambda i: (i, 0, 0)),
        ],
        out_specs=[
            pl.BlockSpec((BN_BLK, DIM), lambda i: (i, 0)),
            pl.BlockSpec((NG, DIM), lambda i: (0, 0)),
            pl.BlockSpec((NG, DIM), lambda i: (0, 0)),
        ],
        out_shape=[
            jax.ShapeDtypeStruct((N, DIM), jnp.float32),
            jax.ShapeDtypeStruct((NG, DIM), jnp.float32),
            jax.ShapeDtypeStruct((NG, DIM), jnp.float32),
        ],
        compiler_params=pltpu.CompilerParams(
            dimension_semantics=("arbitrary",)),
    )(y, st, g, b, batch3)


def _bn_rows(t, g, b):
    mu = jnp.mean(t, axis=0, keepdims=True)
    var = jnp.mean(t * t, axis=0, keepdims=True) - mu * mu
    return g * (t - mu) * jax.lax.rsqrt(var + 1e-5) + b


def _tail_body(p0_ref, p1_ref, p2_ref, cnt_ref, w1_ref, b1_ref, g1_ref,
               bb1_ref, w2_ref, b2_ref, g2_ref, bb2_ref, w3_ref, b3_ref,
               ig_ref, out_ref):
    c = jnp.maximum(cnt_ref[...], 1.0)
    g = jnp.concatenate(
        [p0_ref[...] / c, p1_ref[...] / c, p2_ref[...] / c], axis=1)
    t = jax.lax.dot_general(g, w1_ref[...], (((1,), (0,)), ((), ())),
                            preferred_element_type=jnp.float32) + b1_ref[...]
    t = jnp.maximum(_bn_rows(t, g1_ref[...], bb1_ref[...]), 0.0)
    t = jax.lax.dot_general(t, w2_ref[...], (((1,), (0,)), ((), ())),
                            preferred_element_type=jnp.float32) + b2_ref[...]
    t = jnp.maximum(_bn_rows(t, g2_ref[...], bb2_ref[...]), 0.0)
    ig = ig_ref[0, 0, :]
    oh = (ig[None, :] ==
          jax.lax.broadcasted_iota(jnp.int32, (NS_GRAPH, NG), 0)
          ).astype(jnp.float32)
    ssum = jax.lax.dot_general(oh, t, (((1,), (0,)), ((), ())),
                               preferred_element_type=jnp.float32)
    scnt = jnp.maximum(
        jax.lax.dot_general(oh, jnp.ones_like(t), (((1,), (0,)), ((), ())),
                            preferred_element_type=jnp.float32), 1.0)
    s = ssum / scnt
    out_ref[...] = jax.lax.dot_general(
        s, w3_ref[...], (((1,), (0,)), ((), ())),
        preferred_element_type=jnp.float32) + b3_ref[...]


def _tail(p0, p1, p2, cnt, w1, b1, g1, bb1, w2, b2, g2, bb2, w3, b3, ig3):
    full = lambda shp: pl.BlockSpec(shp, lambda: tuple(0 for _ in shp))
    return pl.pallas_call(
        _tail_body,
        in_specs=[
            full((NG, DIM)), full((NG, DIM)), full((NG, DIM)),
            full((NG, DIM)),
            full((3 * DIM, DIM)), full((1, DIM)), full((1, DIM)),
            full((1, DIM)),
            full((DIM, DIM)), full((1, DIM)), full((1, DIM)), full((1, DIM)),
            full((DIM, 1)), full((1, 1)),
            full((1, 1, NG)),
        ],
        out_specs=full((NS_GRAPH, 1)),
        out_shape=jax.ShapeDtypeStruct((NS_GRAPH, 1), jnp.float32),
    )(p0, p1, p2, cnt, w1, b1, g1, bb1, w2, b2, g2, bb2, w3, b3, ig3)


def _pad2(a, rows, cols):
    return jnp.pad(a, ((0, rows - a.shape[0]), (0, cols - a.shape[1])))


def kernel(x, edge_attr, edge_weight, params, edge_index, batch,
           inter_graph_idx):
    ei = edge_index.astype(jnp.int32)
    src1 = ei[0]
    dst1 = ei[1]
    ew1 = edge_weight
    batch3 = batch.astype(jnp.int32).reshape(N // BN_BLK, 1, BN_BLK)
    ig3 = inter_graph_idx.astype(jnp.int32).reshape(1, 1, NG)

    h = jnp.pad(x, ((0, 0), (0, DIM - x.shape[1])))

    embs = []
    for i in range(3):
        cp = params["convs"][i]
        w_e1 = _pad2(cp["be1"]["W"], 3, DIM)
        b_e1 = _pad2(cp["be1"]["b"][None, :], 1, DIM)
        w_e2 = _pad2(cp["be2"]["W"], DIM, DIM)
        b_e2 = _pad2(cp["be2"]["b"][None, :], 1, DIM)
        embs.append(_edge_mlp(edge_attr, w_e1, b_e1, w_e2, b_e2))

    pools = []
    cnt = None
    for i in range(3):
        cp = params["convs"][i]
        dp = DIM

        agg2 = _mp128(h, embs[i], src1, dst1, ew1)

        w_m1 = _pad2(cp["m1"]["W"], dp, dp)
        b_m1 = _pad2(cp["m1"]["b"][None, :], 1, dp)
        w_m2 = _pad2(cp["m2"]["W"], dp, DIM)
        b_m2 = cp["m2"]["b"][None, :]
        eps = cp["eps"].reshape(1, 1)
        y, st = _node_mlp(h, agg2, eps, w_m1, b_m1, w_m2, b_m2)

        h, pool, cnt_i = _bn_pool(y, st, params["bn_g"][i][None, :],
                                  params["bn_b"][i][None, :], batch3)
        pools.append(pool)
        if cnt is None:
            cnt = cnt_i

    out = _tail(pools[0], pools[1], pools[2], cnt,
                params["fc1"]["W"], params["fc1"]["b"][None, :],
                params["bn1_g"][None, :], params["bn1_b"][None, :],
                params["fc2"]["W"], params["fc2"]["b"][None, :],
                params["bn2_g"][None, :], params["bn2_b"][None, :],
                params["fc3"]["W"], params["fc3"]["b"][None, :],
                ig3)
    return out.reshape(-1)
